# baseline (speedup 1.0000x reference)
import functools
import jax
import jax.numpy as jnp
import numpy as np
from jax import lax
from jax.experimental import pallas as pl
from jax.experimental.pallas import tpu as pltpu
from jax.experimental.pallas import tpu_sc as plsc

N = 10000
L = 30
C = 8
D = 128
E = 320000
EDGE_DIM = 2

_TILE = 1000

_NC = 2
_NS = 16
_NW = _NC * _NS
_EW = E // _NW
_CH = 80
_NCH = _EW // _CH


def _gx_body(x_ref, wx_ref, b_ref, out_ref):
    out_ref[...] = (jnp.dot(x_ref[...], wx_ref[...],
                            preferred_element_type=jnp.float32) + b_ref[...])


def _gx_matmul(xt2, wx, b):
    rows = L * N
    tile = 2000
    return pl.pallas_call(
        _gx_body,
        grid=(rows // tile,),
        in_specs=[
            pl.BlockSpec((tile, C), lambda i: (i, 0)),
            pl.BlockSpec((C, 4 * D), lambda i: (0, 0)),
            pl.BlockSpec((1, 4 * D), lambda i: (0, 0)),
        ],
        out_specs=pl.BlockSpec((tile, 4 * D), lambda i: (i, 0)),
        out_shape=jax.ShapeDtypeStruct((rows, 4 * D), jnp.float32),
        compiler_params=pltpu.CompilerParams(
            dimension_semantics=("parallel",)),
    )(xt2, wx, b)


def _lstm_body(gx_ref, whh_ref, out_ref, h_ref, c_ref):
    t = pl.program_id(1)

    @pl.when(t == 0)
    def _():
        h_ref[...] = jnp.zeros_like(h_ref)
        c_ref[...] = jnp.zeros_like(c_ref)

    g = gx_ref[0] + jnp.dot(h_ref[...], whh_ref[...],
                            preferred_element_type=jnp.float32)
    i = jax.nn.sigmoid(g[:, :D])
    f = jax.nn.sigmoid(g[:, D:2 * D])
    gg = jnp.tanh(g[:, 2 * D:3 * D])
    o = jax.nn.sigmoid(g[:, 3 * D:])
    c_ref[...] = f * c_ref[...] + i * gg
    h_ref[...] = o * jnp.tanh(c_ref[...])

    @pl.when(t == L - 1)
    def _():
        out_ref[...] = h_ref[...]


def _lstm(gx_all, whh):
    return pl.pallas_call(
        _lstm_body,
        grid=(N // _TILE, L),
        in_specs=[
            pl.BlockSpec((1, _TILE, 4 * D), lambda n, t: (t, n, 0)),
            pl.BlockSpec((D, 4 * D), lambda n, t: (0, 0)),
        ],
        out_specs=pl.BlockSpec((_TILE, D), lambda n, t: (n, 0)),
        out_shape=jax.ShapeDtypeStruct((N, D), jnp.float32),
        scratch_shapes=[
            pltpu.VMEM((_TILE, D), jnp.float32),
            pltpu.VMEM((_TILE, D), jnp.float32),
        ],
        compiler_params=pltpu.CompilerParams(
            dimension_semantics=("parallel", "arbitrary")),
    )(gx_all, whh)


def _proj_body(h_ref, wq_ref, wk_ref, wv_ref, ws_ref, bq_ref,
               bk_ref, bv_ref, bs_ref, q_ref, tabs_ref, s_ref):
    h = h_ref[...]
    q_ref[...] = jnp.dot(h, wq_ref[...], preferred_element_type=jnp.float32) + bq_ref[...]
    tabs_ref[:, :D] = jnp.dot(h, wk_ref[...], preferred_element_type=jnp.float32) + bk_ref[...]
    tabs_ref[:, D:] = jnp.dot(h, wv_ref[...], preferred_element_type=jnp.float32) + bv_ref[...]
    s_ref[...] = jnp.dot(h, ws_ref[...], preferred_element_type=jnp.float32) + bs_ref[...]


def _projections(h, wq, wk, wv, ws, bq, bk, bv, bs):
    grid = N // _TILE
    row = lambda i: (i, 0)
    fix = lambda i: (0, 0)
    return pl.pallas_call(
        _proj_body,
        grid=(grid,),
        in_specs=[
            pl.BlockSpec((_TILE, D), row),
            pl.BlockSpec((D, D), fix),
            pl.BlockSpec((D, D), fix),
            pl.BlockSpec((D, D), fix),
            pl.BlockSpec((D, D), fix),
            pl.BlockSpec((1, D), fix),
            pl.BlockSpec((1, D), fix),
            pl.BlockSpec((1, D), fix),
            pl.BlockSpec((1, D), fix),
        ],
        out_specs=[
            pl.BlockSpec((_TILE, D), row),
            pl.BlockSpec((_TILE, 2 * D), row),
            pl.BlockSpec((_TILE, D), row),
        ],
        out_shape=[
            jax.ShapeDtypeStruct((N, D), jnp.float32),
            jax.ShapeDtypeStruct((N, 2 * D), jnp.float32),
            jax.ShapeDtypeStruct((N, D), jnp.float32),
        ],
        compiler_params=pltpu.CompilerParams(
            dimension_semantics=("parallel",)),
    )(h, wq, wk, wv, ws, bq, bk, bv, bs)


def _sc_gather_body(src_hbm, tabs_hbm, dst_hbm, tabd_hbm, gkv_hbm, gq_hbm,
                    idx1, idx2, buf1, buf2, sem1, sem2):
    wid = lax.axis_index("s") * _NC + lax.axis_index("c")
    base = wid * _EW

    def body(j, carry):
        b = base + j * _CH
        pltpu.sync_copy(src_hbm.at[pl.ds(b, _CH)], idx1)
        pltpu.async_copy(tabs_hbm.at[idx1], buf1, sem1).wait()
        pltpu.sync_copy(buf1, gkv_hbm.at[pl.ds(b, _CH)])
        pltpu.sync_copy(dst_hbm.at[pl.ds(b, _CH)], idx2)
        pltpu.async_copy(tabd_hbm.at[idx2], buf2, sem2).wait()
        pltpu.sync_copy(buf2, gq_hbm.at[pl.ds(b, _CH)])
        return carry

    lax.fori_loop(0, _NCH, body, 0)


def _sc_gather(src, tabs, dst, tabd):
    fn = pl.kernel(
        _sc_gather_body,
        out_type=[
            jax.ShapeDtypeStruct((E, 2 * D), jnp.float32),
            jax.ShapeDtypeStruct((E, D), jnp.float32),
        ],
        mesh=plsc.VectorSubcoreMesh(core_axis_name="c", subcore_axis_name="s"),
        scratch_types=[
            pltpu.VMEM((_CH,), jnp.int32),
            pltpu.VMEM((_CH,), jnp.int32),
            pltpu.VMEM((_CH, 2 * D), jnp.float32),
            pltpu.VMEM((_CH, D), jnp.float32),
            pltpu.SemaphoreType.DMA,
            pltpu.SemaphoreType.DMA,
        ],
    )
    return fn(src, tabs, dst, tabd)


def _sc_scatter_body(pay_hbm, dst_hbm, ea_hbm, zrow_hbm, zs_hbm,
                     outrow_hbm, outs_hbm, idx_v, buf_v, ea_v, s_tab, shared):
    cid = lax.axis_index("c")
    sid = lax.axis_index("s")
    wid = sid * _NC + cid
    base = wid * _EW

    pltpu.sync_copy(zs_hbm, s_tab)

    @pl.when(sid == 0)
    def _():
        pltpu.sync_copy(zrow_hbm, shared)

    plsc.subcore_barrier()

    def body(j, carry):
        b = base + j * _CH
        pltpu.sync_copy(dst_hbm.at[pl.ds(b, _CH)], idx_v)
        pltpu.sync_copy(pay_hbm.at[pl.ds(b, _CH)], buf_v)
        pltpu.sync_copy(ea_hbm.at[pl.ds(b, _CH)], ea_v)
        pltpu.sync_copy(buf_v, shared.at[idx_v], add=True)
        for j2 in range(_CH // 16):
            plsc.addupdate_scatter(
                s_tab, [idx_v[pl.ds(j2 * 16, 16)]], ea_v[pl.ds(j2 * 16, 16)])
        return carry

    lax.fori_loop(0, _NCH, body, 0)

    pltpu.sync_copy(s_tab, outs_hbm.at[pl.ds(wid * N, N)])
    plsc.subcore_barrier()

    @pl.when(sid < _NS - 1)
    def _():
        pltpu.sync_copy(shared.at[pl.ds(sid * 624, 624)],
                        outrow_hbm.at[cid, pl.ds(sid * 624, 624)])

    @pl.when(sid == _NS - 1)
    def _():
        pltpu.sync_copy(shared.at[pl.ds(15 * 624, N - 15 * 624)],
                        outrow_hbm.at[cid, pl.ds(15 * 624, N - 15 * 624)])


def _sc_scatter(payload, dst, ea, zrow, zs):
    fn = pl.kernel(
        _sc_scatter_body,
        out_type=[
            jax.ShapeDtypeStruct((_NC, N, D), jnp.float32),
            jax.ShapeDtypeStruct((_NW * N,), jnp.float32),
        ],
        mesh=plsc.VectorSubcoreMesh(core_axis_name="c", subcore_axis_name="s"),
        scratch_types=[
            pltpu.VMEM((_CH,), jnp.int32),
            pltpu.VMEM((_CH, D), jnp.float32),
            pltpu.VMEM((_CH,), jnp.float32),
            pltpu.VMEM((N,), jnp.float32),
            pltpu.VMEM_SHARED((N, D), jnp.float32),
        ],
        compiler_params=pltpu.CompilerParams(needs_layout_passes=False),
    )
    return fn(payload, dst, ea, zrow, zs)


_ETILE = 2000
_NEB = E // _ETILE


def _alpha_body(gq_ref, gkv_ref, attr_ref, wep_ref, a_ref, bm_ref):
    qd = gq_ref[...]
    ks = gkv_ref[:, :D]
    qe = jnp.dot(qd, wep_ref[...], preferred_element_type=jnp.float32)
    al = (jnp.sum(qd * ks, axis=1, keepdims=True)
          + jnp.sum(qe * attr_ref[...], axis=1, keepdims=True)
          ) * (1.0 / np.sqrt(float(D)))
    a_ref[...] = al
    bm_ref[...] = jnp.max(al).reshape(1, 1, 1)


def _alpha(gq, gkv, attrp, wep):
    row = lambda i: (i, 0)
    return pl.pallas_call(
        _alpha_body,
        grid=(_NEB,),
        in_specs=[
            pl.BlockSpec((_ETILE, D), row),
            pl.BlockSpec((_ETILE, 2 * D), row),
            pl.BlockSpec((_ETILE, 8), row),
            pl.BlockSpec((D, 8), lambda i: (0, 0)),
        ],
        out_specs=[
            pl.BlockSpec((_ETILE, 1), row),
            pl.BlockSpec((1, 1, 1), lambda i: (i, 0, 0)),
        ],
        out_shape=[
            jax.ShapeDtypeStruct((E, 1), jnp.float32),
            jax.ShapeDtypeStruct((_NEB, 1, 1), jnp.float32),
        ],
        compiler_params=pltpu.CompilerParams(
            dimension_semantics=("parallel",)),
    )(gq, gkv, attrp, wep)


def _payload_body(a_ref, bm_ref, gkv_ref, attr_ref, wet_ref, p_ref, ea_ref):
    kk = jnp.max(bm_ref[...])
    ea = jnp.exp(a_ref[...] - kk)
    vj = gkv_ref[:, D:] + jnp.dot(attr_ref[...], wet_ref[...],
                                  preferred_element_type=jnp.float32)
    p_ref[...] = vj * ea
    ea_ref[...] = ea


def _payload(alpha, bmax, gkv, attrp, wet):
    row = lambda i: (i, 0)
    return pl.pallas_call(
        _payload_body,
        grid=(_NEB,),
        in_specs=[
            pl.BlockSpec((_ETILE, 1), row),
            pl.BlockSpec((_NEB, 1, 1), lambda i: (0, 0, 0)),
            pl.BlockSpec((_ETILE, 2 * D), row),
            pl.BlockSpec((_ETILE, 8), row),
            pl.BlockSpec((8, D), lambda i: (0, 0)),
        ],
        out_specs=[
            pl.BlockSpec((_ETILE, D), row),
            pl.BlockSpec((_ETILE, 1), row),
        ],
        out_shape=[
            jax.ShapeDtypeStruct((E, D), jnp.float32),
            jax.ShapeDtypeStruct((E, 1), jnp.float32),
        ],
        compiler_params=pltpu.CompilerParams(
            dimension_semantics=("parallel",)),
    )(alpha, bmax, gkv, attrp, wet)


def _combine_body(a0_ref, a1_ref, sp_ref, skip_ref, hprev_ref, g_ref,
                  b_ref, out_ref):
    accv = a0_ref[0] + a1_ref[0]
    ones = jnp.ones((_NW, 1), jnp.float32)
    s_col = lax.dot_general(sp_ref[0], ones, (((0,), (0,)), ((), ())),
                            preferred_element_type=jnp.float32)
    msg = accv / (s_col + 1e-16)
    y = hprev_ref[...] + msg + skip_ref[...]
    mu = jnp.mean(y, axis=-1, keepdims=True)
    var = jnp.mean((y - mu) ** 2, axis=-1, keepdims=True)
    out_ref[...] = (y - mu) / jnp.sqrt(var + 1e-5) * g_ref[...] + b_ref[...]


def _combine(acc, spart, skip, hprev, g, b):
    grid = N // _TILE
    row = lambda i: (i, 0)
    fix = lambda i: (0, 0)
    return pl.pallas_call(
        _combine_body,
        grid=(grid,),
        in_specs=[
            pl.BlockSpec((1, _TILE, D), lambda i: (0, i, 0)),
            pl.BlockSpec((1, _TILE, D), lambda i: (1, i, 0)),
            pl.BlockSpec((1, _NW, _TILE), lambda i: (i, 0, 0)),
            pl.BlockSpec((_TILE, D), row),
            pl.BlockSpec((_TILE, D), row),
            pl.BlockSpec((1, D), fix),
            pl.BlockSpec((1, D), fix),
        ],
        out_specs=pl.BlockSpec((_TILE, D), row),
        out_shape=jax.ShapeDtypeStruct((N, D), jnp.float32),
        compiler_params=pltpu.CompilerParams(
            dimension_semantics=("parallel",)),
    )(acc, acc, spart, skip, hprev, g, b)


def _head_body(h_ref, w1_ref, b1_ref, w2_ref, b2_ref, out_ref):
    hid = jax.nn.relu(
        jnp.dot(h_ref[...], w1_ref[...], preferred_element_type=jnp.float32)
        + b1_ref[...])
    w_raw = jnp.dot(hid, w2_ref[...], preferred_element_type=jnp.float32) + b2_ref[0, 0]
    denom = jnp.sum(jnp.abs(w_raw[:, 0:1]))
    out_ref[...] = w_raw / denom


def _head(h, w1, b1, w2, b2):
    return pl.pallas_call(
        _head_body,
        in_specs=[
            pl.BlockSpec((N, D), lambda: (0, 0)),
            pl.BlockSpec((D, D // 2), lambda: (0, 0)),
            pl.BlockSpec((1, D // 2), lambda: (0, 0)),
            pl.BlockSpec((D // 2, 8), lambda: (0, 0)),
            pl.BlockSpec((1, 1), lambda: (0, 0)),
        ],
        out_specs=pl.BlockSpec((N, 8), lambda: (0, 0)),
        out_shape=jax.ShapeDtypeStruct((N, 8), jnp.float32),
    )(h, w1, b1, w2, b2)


def kernel(x, edge_index, edge_attr, params):
    p = params
    src = edge_index[0].astype(jnp.int32)
    dst = edge_index[1].astype(jnp.int32)
    attrp = jnp.pad(edge_attr, ((0, 0), (0, 8 - EDGE_DIM)))
    zrow = jnp.zeros((N, D), jnp.float32)
    zs = jnp.zeros((N,), jnp.float32)

    w_x = p['W_ih'] @ p['W_in']
    b_all = p['b_ih'] + p['b_hh'] + p['W_ih'] @ p['b_in']
    xt2 = jnp.swapaxes(x, 0, 1).reshape(L * N, C)
    gx_all = _gx_matmul(xt2, w_x.T, b_all.reshape(1, -1)).reshape(L, N, 4 * D)
    h = _lstm(gx_all, p['W_hh'].T)

    for l in range(2):
        we = p['We%d' % l]
        wep = jnp.zeros((D, 8), jnp.float32).at[:, :EDGE_DIM].set(we)
        wet = jnp.zeros((8, D), jnp.float32).at[:EDGE_DIM, :].set(we.T)
        tabd, tabs, skip = _projections(
            h, p['Wq%d' % l].T, p['Wk%d' % l].T, p['Wv%d' % l].T,
            p['Wskip%d' % l].T,
            p['bq%d' % l].reshape(1, -1), p['bk%d' % l].reshape(1, -1),
            p['bv%d' % l].reshape(1, -1), p['bskip%d' % l].reshape(1, -1))

        gkv, gq = _sc_gather(src, tabs, dst, tabd)
        alpha, bmax = _alpha(gq, gkv, attrp, wep)
        payload, ea = _payload(alpha, bmax, gkv, attrp, wet)
        acc, spart = _sc_scatter(payload, dst, ea.reshape(E), zrow, zs)
        sp3 = spart.reshape(_NW, N // _TILE, _TILE).swapaxes(0, 1)

        h = _combine(acc, sp3, skip, h,
                     p['ln_g%d' % l].reshape(1, -1),
                     p['ln_b%d' % l].reshape(1, -1))

    out = _head(h, p['W1'].T, p['b1'].reshape(1, -1),
                jnp.zeros((D // 2, 8), jnp.float32).at[:, 0].set(p['W2'][0]),
                p['b2'].reshape(1, 1))
    return out[:, 0]


# speedup vs baseline: 1.1500x; 1.1500x over previous
import functools
import jax
import jax.numpy as jnp
import numpy as np
from jax import lax
from jax.experimental import pallas as pl
from jax.experimental.pallas import tpu as pltpu
from jax.experimental.pallas import tpu_sc as plsc

N = 10000
L = 30
C = 8
D = 128
E = 320000
EDGE_DIM = 2

_TILE = 1000

_NC = 2
_NS = 16
_NW = _NC * _NS
_EW = E // _NW
_CH = 80
_NCH = _EW // _CH


def _gx_body(x_ref, wx_ref, b_ref, out_ref):
    out_ref[...] = (jnp.dot(x_ref[...], wx_ref[...],
                            preferred_element_type=jnp.float32) + b_ref[...])


def _gx_matmul(xt2, wx, b):
    rows = L * N
    tile = 2000
    return pl.pallas_call(
        _gx_body,
        grid=(rows // tile,),
        in_specs=[
            pl.BlockSpec((tile, C), lambda i: (i, 0)),
            pl.BlockSpec((C, 4 * D), lambda i: (0, 0)),
            pl.BlockSpec((1, 4 * D), lambda i: (0, 0)),
        ],
        out_specs=pl.BlockSpec((tile, 4 * D), lambda i: (i, 0)),
        out_shape=jax.ShapeDtypeStruct((rows, 4 * D), jnp.float32),
        compiler_params=pltpu.CompilerParams(
            dimension_semantics=("parallel",)),
    )(xt2, wx, b)


def _lstm_body(gx_ref, whh_ref, out_ref, h_ref, c_ref):
    t = pl.program_id(1)

    @pl.when(t == 0)
    def _():
        h_ref[...] = jnp.zeros_like(h_ref)
        c_ref[...] = jnp.zeros_like(c_ref)

    g = gx_ref[0] + jnp.dot(h_ref[...], whh_ref[...],
                            preferred_element_type=jnp.float32)
    i = jax.nn.sigmoid(g[:, :D])
    f = jax.nn.sigmoid(g[:, D:2 * D])
    gg = jnp.tanh(g[:, 2 * D:3 * D])
    o = jax.nn.sigmoid(g[:, 3 * D:])
    c_ref[...] = f * c_ref[...] + i * gg
    h_ref[...] = o * jnp.tanh(c_ref[...])

    @pl.when(t == L - 1)
    def _():
        out_ref[...] = h_ref[...]


def _lstm(gx_all, whh):
    return pl.pallas_call(
        _lstm_body,
        grid=(N // _TILE, L),
        in_specs=[
            pl.BlockSpec((1, _TILE, 4 * D), lambda n, t: (t, n, 0)),
            pl.BlockSpec((D, 4 * D), lambda n, t: (0, 0)),
        ],
        out_specs=pl.BlockSpec((_TILE, D), lambda n, t: (n, 0)),
        out_shape=jax.ShapeDtypeStruct((N, D), jnp.float32),
        scratch_shapes=[
            pltpu.VMEM((_TILE, D), jnp.float32),
            pltpu.VMEM((_TILE, D), jnp.float32),
        ],
        compiler_params=pltpu.CompilerParams(
            dimension_semantics=("parallel", "arbitrary")),
    )(gx_all, whh)


def _proj_body(h_ref, wq_ref, wk_ref, wv_ref, ws_ref, bq_ref,
               bk_ref, bv_ref, bs_ref, q_ref, tabs_ref, s_ref):
    h = h_ref[...]
    q_ref[...] = jnp.dot(h, wq_ref[...], preferred_element_type=jnp.float32) + bq_ref[...]
    tabs_ref[:, :D] = jnp.dot(h, wk_ref[...], preferred_element_type=jnp.float32) + bk_ref[...]
    tabs_ref[:, D:] = jnp.dot(h, wv_ref[...], preferred_element_type=jnp.float32) + bv_ref[...]
    s_ref[...] = jnp.dot(h, ws_ref[...], preferred_element_type=jnp.float32) + bs_ref[...]


def _projections(h, wq, wk, wv, ws, bq, bk, bv, bs):
    grid = N // _TILE
    row = lambda i: (i, 0)
    fix = lambda i: (0, 0)
    return pl.pallas_call(
        _proj_body,
        grid=(grid,),
        in_specs=[
            pl.BlockSpec((_TILE, D), row),
            pl.BlockSpec((D, D), fix),
            pl.BlockSpec((D, D), fix),
            pl.BlockSpec((D, D), fix),
            pl.BlockSpec((D, D), fix),
            pl.BlockSpec((1, D), fix),
            pl.BlockSpec((1, D), fix),
            pl.BlockSpec((1, D), fix),
            pl.BlockSpec((1, D), fix),
        ],
        out_specs=[
            pl.BlockSpec((_TILE, D), row),
            pl.BlockSpec((_TILE, 2 * D), row),
            pl.BlockSpec((_TILE, D), row),
        ],
        out_shape=[
            jax.ShapeDtypeStruct((N, D), jnp.float32),
            jax.ShapeDtypeStruct((N, 2 * D), jnp.float32),
            jax.ShapeDtypeStruct((N, D), jnp.float32),
        ],
        compiler_params=pltpu.CompilerParams(
            dimension_semantics=("parallel",)),
    )(h, wq, wk, wv, ws, bq, bk, bv, bs)


_GCH = 128
_GN = _EW // _GCH
_GTAIL = _EW - _GN * _GCH


def _sc_gather_body(src_hbm, tabs_hbm, dst_hbm, tabd_hbm, gkv_hbm, gq_hbm,
                    idxs, idxd, b1a, b2a, b1b, b2b, gsa, gsb, wsa, wsb):
    wid = lax.axis_index("s") * _NC + lax.axis_index("c")
    base = wid * _EW
    pltpu.sync_copy(src_hbm.at[pl.ds(base, _EW)], idxs)
    pltpu.sync_copy(dst_hbm.at[pl.ds(base, _EW)], idxd)

    def issue(j, b1, b2, gs):
        off = j * _GCH
        pltpu.async_copy(tabs_hbm.at[idxs.at[pl.ds(off, _GCH)]], b1, gs)
        pltpu.async_copy(tabd_hbm.at[idxd.at[pl.ds(off, _GCH)]], b2, gs)

    def wait_g(b1, b2, gs):
        pltpu.make_async_copy(tabs_hbm.at[pl.ds(0, _GCH)], b1, gs).wait()
        pltpu.make_async_copy(tabd_hbm.at[pl.ds(0, _GCH)], b2, gs).wait()

    def wb(j, b1, b2, ws):
        b = base + j * _GCH
        pltpu.async_copy(b1, gkv_hbm.at[pl.ds(b, _GCH)], ws)
        pltpu.async_copy(b2, gq_hbm.at[pl.ds(b, _GCH)], ws)

    def wait_wb(b1, b2, ws):
        pltpu.make_async_copy(b1, gkv_hbm.at[pl.ds(0, _GCH)], ws).wait()
        pltpu.make_async_copy(b2, gq_hbm.at[pl.ds(0, _GCH)], ws).wait()

    issue(0, b1a, b2a, gsa)

    def body(j, carry):
        for pp in range(2):
            if pp == 0:
                b1, b2, gs, ws = b1a, b2a, gsa, wsa
                ob1, ob2, ogs, ows = b1b, b2b, gsb, wsb
            else:
                b1, b2, gs, ws = b1b, b2b, gsb, wsb
                ob1, ob2, ogs, ows = b1a, b2a, gsa, wsa

            @pl.when(lax.rem(j, 2) == pp)
            def _():
                wait_g(b1, b2, gs)

                @pl.when(j >= 1)
                def _():
                    wait_wb(ob1, ob2, ows)

                @pl.when(j + 1 < _GN)
                def _():
                    issue(j + 1, ob1, ob2, ogs)

                wb(j, b1, b2, ws)
        return carry

    lax.fori_loop(0, _GN, body, 0)
    if (_GN - 1) % 2 == 0:
        wait_wb(b1a, b2a, wsa)
    else:
        wait_wb(b1b, b2b, wsb)

    toff = _GN * _GCH
    pltpu.async_copy(tabs_hbm.at[idxs.at[pl.ds(toff, _GTAIL)]],
                     b1a.at[pl.ds(0, _GTAIL)], gsa)
    pltpu.async_copy(tabd_hbm.at[idxd.at[pl.ds(toff, _GTAIL)]],
                     b2a.at[pl.ds(0, _GTAIL)], gsa)
    pltpu.make_async_copy(tabs_hbm.at[pl.ds(0, _GTAIL)],
                          b1a.at[pl.ds(0, _GTAIL)], gsa).wait()
    pltpu.make_async_copy(tabd_hbm.at[pl.ds(0, _GTAIL)],
                          b2a.at[pl.ds(0, _GTAIL)], gsa).wait()
    pltpu.sync_copy(b1a.at[pl.ds(0, _GTAIL)],
                    gkv_hbm.at[pl.ds(base + toff, _GTAIL)])
    pltpu.sync_copy(b2a.at[pl.ds(0, _GTAIL)],
                    gq_hbm.at[pl.ds(base + toff, _GTAIL)])


def _sc_gather(src, tabs, dst, tabd):
    fn = pl.kernel(
        _sc_gather_body,
        out_type=[
            jax.ShapeDtypeStruct((E, 2 * D), jnp.float32),
            jax.ShapeDtypeStruct((E, D), jnp.float32),
        ],
        mesh=plsc.VectorSubcoreMesh(core_axis_name="c", subcore_axis_name="s"),
        scratch_types=[
            pltpu.VMEM((_EW,), jnp.int32),
            pltpu.VMEM((_EW,), jnp.int32),
            pltpu.VMEM((_GCH, 2 * D), jnp.float32),
            pltpu.VMEM((_GCH, D), jnp.float32),
            pltpu.VMEM((_GCH, 2 * D), jnp.float32),
            pltpu.VMEM((_GCH, D), jnp.float32),
            pltpu.SemaphoreType.DMA,
            pltpu.SemaphoreType.DMA,
            pltpu.SemaphoreType.DMA,
            pltpu.SemaphoreType.DMA,
        ],
        compiler_params=pltpu.CompilerParams(needs_layout_passes=False),
    )
    return fn(src, tabs, dst, tabd)


def _sc_scatter_body(pay_hbm, dst_hbm, ea_hbm, zrow_hbm, zs_hbm,
                     outrow_hbm, outs_hbm, idx_v, buf_v, ea_v, s_tab, shared):
    cid = lax.axis_index("c")
    sid = lax.axis_index("s")
    wid = sid * _NC + cid
    base = wid * _EW

    pltpu.sync_copy(zs_hbm, s_tab)

    @pl.when(sid == 0)
    def _():
        pltpu.sync_copy(zrow_hbm, shared)

    plsc.subcore_barrier()

    def body(j, carry):
        b = base + j * _CH
        pltpu.sync_copy(dst_hbm.at[pl.ds(b, _CH)], idx_v)
        pltpu.sync_copy(pay_hbm.at[pl.ds(b, _CH)], buf_v)
        pltpu.sync_copy(ea_hbm.at[pl.ds(b, _CH)], ea_v)
        pltpu.sync_copy(buf_v, shared.at[idx_v], add=True)
        for j2 in range(_CH // 16):
            plsc.addupdate_scatter(
                s_tab, [idx_v[pl.ds(j2 * 16, 16)]], ea_v[pl.ds(j2 * 16, 16)])
        return carry

    lax.fori_loop(0, _NCH, body, 0)

    pltpu.sync_copy(s_tab, outs_hbm.at[pl.ds(wid * N, N)])
    plsc.subcore_barrier()

    @pl.when(sid < _NS - 1)
    def _():
        pltpu.sync_copy(shared.at[pl.ds(sid * 624, 624)],
                        outrow_hbm.at[cid, pl.ds(sid * 624, 624)])

    @pl.when(sid == _NS - 1)
    def _():
        pltpu.sync_copy(shared.at[pl.ds(15 * 624, N - 15 * 624)],
                        outrow_hbm.at[cid, pl.ds(15 * 624, N - 15 * 624)])


def _sc_scatter(payload, dst, ea, zrow, zs):
    fn = pl.kernel(
        _sc_scatter_body,
        out_type=[
            jax.ShapeDtypeStruct((_NC, N, D), jnp.float32),
            jax.ShapeDtypeStruct((_NW * N,), jnp.float32),
        ],
        mesh=plsc.VectorSubcoreMesh(core_axis_name="c", subcore_axis_name="s"),
        scratch_types=[
            pltpu.VMEM((_CH,), jnp.int32),
            pltpu.VMEM((_CH, D), jnp.float32),
            pltpu.VMEM((_CH,), jnp.float32),
            pltpu.VMEM((N,), jnp.float32),
            pltpu.VMEM_SHARED((N, D), jnp.float32),
        ],
        compiler_params=pltpu.CompilerParams(needs_layout_passes=False),
    )
    return fn(payload, dst, ea, zrow, zs)


_ETILE = 2000
_NEB = E // _ETILE


def _alpha_body(gq_ref, gkv_ref, attr_ref, wep_ref, a_ref, bm_ref):
    qd = gq_ref[...]
    ks = gkv_ref[:, :D]
    qe = jnp.dot(qd, wep_ref[...], preferred_element_type=jnp.float32)
    al = (jnp.sum(qd * ks, axis=1, keepdims=True)
          + jnp.sum(qe * attr_ref[...], axis=1, keepdims=True)
          ) * (1.0 / np.sqrt(float(D)))
    a_ref[...] = al
    bm_ref[...] = jnp.max(al).reshape(1, 1, 1)


def _alpha(gq, gkv, attrp, wep):
    row = lambda i: (i, 0)
    return pl.pallas_call(
        _alpha_body,
        grid=(_NEB,),
        in_specs=[
            pl.BlockSpec((_ETILE, D), row),
            pl.BlockSpec((_ETILE, 2 * D), row),
            pl.BlockSpec((_ETILE, 8), row),
            pl.BlockSpec((D, 8), lambda i: (0, 0)),
        ],
        out_specs=[
            pl.BlockSpec((_ETILE, 1), row),
            pl.BlockSpec((1, 1, 1), lambda i: (i, 0, 0)),
        ],
        out_shape=[
            jax.ShapeDtypeStruct((E, 1), jnp.float32),
            jax.ShapeDtypeStruct((_NEB, 1, 1), jnp.float32),
        ],
        compiler_params=pltpu.CompilerParams(
            dimension_semantics=("parallel",)),
    )(gq, gkv, attrp, wep)


def _payload_body(a_ref, bm_ref, gkv_ref, attr_ref, wet_ref, p_ref, ea_ref):
    kk = jnp.max(bm_ref[...])
    ea = jnp.exp(a_ref[...] - kk)
    vj = gkv_ref[:, D:] + jnp.dot(attr_ref[...], wet_ref[...],
                                  preferred_element_type=jnp.float32)
    p_ref[...] = vj * ea
    ea_ref[...] = ea


def _payload(alpha, bmax, gkv, attrp, wet):
    row = lambda i: (i, 0)
    return pl.pallas_call(
        _payload_body,
        grid=(_NEB,),
        in_specs=[
            pl.BlockSpec((_ETILE, 1), row),
            pl.BlockSpec((_NEB, 1, 1), lambda i: (0, 0, 0)),
            pl.BlockSpec((_ETILE, 2 * D), row),
            pl.BlockSpec((_ETILE, 8), row),
            pl.BlockSpec((8, D), lambda i: (0, 0)),
        ],
        out_specs=[
            pl.BlockSpec((_ETILE, D), row),
            pl.BlockSpec((_ETILE, 1), row),
        ],
        out_shape=[
            jax.ShapeDtypeStruct((E, D), jnp.float32),
            jax.ShapeDtypeStruct((E, 1), jnp.float32),
        ],
        compiler_params=pltpu.CompilerParams(
            dimension_semantics=("parallel",)),
    )(alpha, bmax, gkv, attrp, wet)


def _combine_body(a0_ref, a1_ref, sp_ref, skip_ref, hprev_ref, g_ref,
                  b_ref, out_ref):
    accv = a0_ref[0] + a1_ref[0]
    ones = jnp.ones((_NW, 1), jnp.float32)
    s_col = lax.dot_general(sp_ref[0], ones, (((0,), (0,)), ((), ())),
                            preferred_element_type=jnp.float32)
    msg = accv / (s_col + 1e-16)
    y = hprev_ref[...] + msg + skip_ref[...]
    mu = jnp.mean(y, axis=-1, keepdims=True)
    var = jnp.mean((y - mu) ** 2, axis=-1, keepdims=True)
    out_ref[...] = (y - mu) / jnp.sqrt(var + 1e-5) * g_ref[...] + b_ref[...]


def _combine(acc, spart, skip, hprev, g, b):
    grid = N // _TILE
    row = lambda i: (i, 0)
    fix = lambda i: (0, 0)
    return pl.pallas_call(
        _combine_body,
        grid=(grid,),
        in_specs=[
            pl.BlockSpec((1, _TILE, D), lambda i: (0, i, 0)),
            pl.BlockSpec((1, _TILE, D), lambda i: (1, i, 0)),
            pl.BlockSpec((1, _NW, _TILE), lambda i: (i, 0, 0)),
            pl.BlockSpec((_TILE, D), row),
            pl.BlockSpec((_TILE, D), row),
            pl.BlockSpec((1, D), fix),
            pl.BlockSpec((1, D), fix),
        ],
        out_specs=pl.BlockSpec((_TILE, D), row),
        out_shape=jax.ShapeDtypeStruct((N, D), jnp.float32),
        compiler_params=pltpu.CompilerParams(
            dimension_semantics=("parallel",)),
    )(acc, acc, spart, skip, hprev, g, b)


def _head_body(h_ref, w1_ref, b1_ref, w2_ref, b2_ref, out_ref):
    hid = jax.nn.relu(
        jnp.dot(h_ref[...], w1_ref[...], preferred_element_type=jnp.float32)
        + b1_ref[...])
    w_raw = jnp.dot(hid, w2_ref[...], preferred_element_type=jnp.float32) + b2_ref[0, 0]
    denom = jnp.sum(jnp.abs(w_raw[:, 0:1]))
    out_ref[...] = w_raw / denom


def _head(h, w1, b1, w2, b2):
    return pl.pallas_call(
        _head_body,
        in_specs=[
            pl.BlockSpec((N, D), lambda: (0, 0)),
            pl.BlockSpec((D, D // 2), lambda: (0, 0)),
            pl.BlockSpec((1, D // 2), lambda: (0, 0)),
            pl.BlockSpec((D // 2, 8), lambda: (0, 0)),
            pl.BlockSpec((1, 1), lambda: (0, 0)),
        ],
        out_specs=pl.BlockSpec((N, 8), lambda: (0, 0)),
        out_shape=jax.ShapeDtypeStruct((N, 8), jnp.float32),
    )(h, w1, b1, w2, b2)


def kernel(x, edge_index, edge_attr, params):
    p = params
    src = edge_index[0].astype(jnp.int32)
    dst = edge_index[1].astype(jnp.int32)
    attrp = jnp.pad(edge_attr, ((0, 0), (0, 8 - EDGE_DIM)))
    zrow = jnp.zeros((N, D), jnp.float32)
    zs = jnp.zeros((N,), jnp.float32)

    w_x = p['W_ih'] @ p['W_in']
    b_all = p['b_ih'] + p['b_hh'] + p['W_ih'] @ p['b_in']
    xt2 = jnp.swapaxes(x, 0, 1).reshape(L * N, C)
    gx_all = _gx_matmul(xt2, w_x.T, b_all.reshape(1, -1)).reshape(L, N, 4 * D)
    h = _lstm(gx_all, p['W_hh'].T)

    for l in range(2):
        we = p['We%d' % l]
        wep = jnp.zeros((D, 8), jnp.float32).at[:, :EDGE_DIM].set(we)
        wet = jnp.zeros((8, D), jnp.float32).at[:EDGE_DIM, :].set(we.T)
        tabd, tabs, skip = _projections(
            h, p['Wq%d' % l].T, p['Wk%d' % l].T, p['Wv%d' % l].T,
            p['Wskip%d' % l].T,
            p['bq%d' % l].reshape(1, -1), p['bk%d' % l].reshape(1, -1),
            p['bv%d' % l].reshape(1, -1), p['bskip%d' % l].reshape(1, -1))

        gkv, gq = _sc_gather(src, tabs, dst, tabd)
        alpha, bmax = _alpha(gq, gkv, attrp, wep)
        payload, ea = _payload(alpha, bmax, gkv, attrp, wet)
        acc, spart = _sc_scatter(payload, dst, ea.reshape(E), zrow, zs)
        sp3 = spart.reshape(_NW, N // _TILE, _TILE).swapaxes(0, 1)

        h = _combine(acc, sp3, skip, h,
                     p['ln_g%d' % l].reshape(1, -1),
                     p['ln_b%d' % l].reshape(1, -1))

    out = _head(h, p['W1'].T, p['b1'].reshape(1, -1),
                jnp.zeros((D // 2, 8), jnp.float32).at[:, 0].set(p['W2'][0]),
                p['b2'].reshape(1, 1))
    return out[:, 0]


# speedup vs baseline: 1.2634x; 1.0986x over previous
import functools
import jax
import jax.numpy as jnp
import numpy as np
from jax import lax
from jax.experimental import pallas as pl
from jax.experimental.pallas import tpu as pltpu
from jax.experimental.pallas import tpu_sc as plsc

N = 10000
L = 30
C = 8
D = 128
E = 320000
EDGE_DIM = 2

_TILE = 1000

_NC = 2
_NS = 16
_NW = _NC * _NS
_EW = E // _NW
_CH = 80
_NCH = _EW // _CH


def _gx_body(x_ref, wx_ref, b_ref, out_ref):
    out_ref[...] = (jnp.dot(x_ref[...], wx_ref[...],
                            preferred_element_type=jnp.float32) + b_ref[...])


def _gx_matmul(xt2, wx, b):
    rows = L * N
    tile = 2000
    return pl.pallas_call(
        _gx_body,
        grid=(rows // tile,),
        in_specs=[
            pl.BlockSpec((tile, C), lambda i: (i, 0)),
            pl.BlockSpec((C, 4 * D), lambda i: (0, 0)),
            pl.BlockSpec((1, 4 * D), lambda i: (0, 0)),
        ],
        out_specs=pl.BlockSpec((tile, 4 * D), lambda i: (i, 0)),
        out_shape=jax.ShapeDtypeStruct((rows, 4 * D), jnp.float32),
        compiler_params=pltpu.CompilerParams(
            dimension_semantics=("parallel",)),
    )(xt2, wx, b)


def _lstm_body(gx_ref, whh_ref, out_ref, h_ref, c_ref):
    t = pl.program_id(1)

    @pl.when(t == 0)
    def _():
        h_ref[...] = jnp.zeros_like(h_ref)
        c_ref[...] = jnp.zeros_like(c_ref)

    g = gx_ref[0] + jnp.dot(h_ref[...], whh_ref[...],
                            preferred_element_type=jnp.float32)
    i = jax.nn.sigmoid(g[:, :D])
    f = jax.nn.sigmoid(g[:, D:2 * D])
    gg = jnp.tanh(g[:, 2 * D:3 * D])
    o = jax.nn.sigmoid(g[:, 3 * D:])
    c_ref[...] = f * c_ref[...] + i * gg
    h_ref[...] = o * jnp.tanh(c_ref[...])

    @pl.when(t == L - 1)
    def _():
        out_ref[...] = h_ref[...]


def _lstm(gx_all, whh):
    return pl.pallas_call(
        _lstm_body,
        grid=(N // _TILE, L),
        in_specs=[
            pl.BlockSpec((1, _TILE, 4 * D), lambda n, t: (t, n, 0)),
            pl.BlockSpec((D, 4 * D), lambda n, t: (0, 0)),
        ],
        out_specs=pl.BlockSpec((_TILE, D), lambda n, t: (n, 0)),
        out_shape=jax.ShapeDtypeStruct((N, D), jnp.float32),
        scratch_shapes=[
            pltpu.VMEM((_TILE, D), jnp.float32),
            pltpu.VMEM((_TILE, D), jnp.float32),
        ],
        compiler_params=pltpu.CompilerParams(
            dimension_semantics=("parallel", "arbitrary")),
    )(gx_all, whh)


def _proj_body(h_ref, wq_ref, wk_ref, wv_ref, ws_ref, bq_ref,
               bk_ref, bv_ref, bs_ref, q_ref, tabs_ref, s_ref):
    h = h_ref[...]
    q_ref[...] = jnp.dot(h, wq_ref[...], preferred_element_type=jnp.float32) + bq_ref[...]
    tabs_ref[:, :D] = jnp.dot(h, wk_ref[...], preferred_element_type=jnp.float32) + bk_ref[...]
    tabs_ref[:, D:] = jnp.dot(h, wv_ref[...], preferred_element_type=jnp.float32) + bv_ref[...]
    s_ref[...] = jnp.dot(h, ws_ref[...], preferred_element_type=jnp.float32) + bs_ref[...]


def _projections(h, wq, wk, wv, ws, bq, bk, bv, bs):
    grid = N // _TILE
    row = lambda i: (i, 0)
    fix = lambda i: (0, 0)
    return pl.pallas_call(
        _proj_body,
        grid=(grid,),
        in_specs=[
            pl.BlockSpec((_TILE, D), row),
            pl.BlockSpec((D, D), fix),
            pl.BlockSpec((D, D), fix),
            pl.BlockSpec((D, D), fix),
            pl.BlockSpec((D, D), fix),
            pl.BlockSpec((1, D), fix),
            pl.BlockSpec((1, D), fix),
            pl.BlockSpec((1, D), fix),
            pl.BlockSpec((1, D), fix),
        ],
        out_specs=[
            pl.BlockSpec((_TILE, D), row),
            pl.BlockSpec((_TILE, 2 * D), row),
            pl.BlockSpec((_TILE, D), row),
        ],
        out_shape=[
            jax.ShapeDtypeStruct((N, D), jnp.float32),
            jax.ShapeDtypeStruct((N, 2 * D), jnp.float32),
            jax.ShapeDtypeStruct((N, D), jnp.float32),
        ],
        compiler_params=pltpu.CompilerParams(
            dimension_semantics=("parallel",)),
    )(h, wq, wk, wv, ws, bq, bk, bv, bs)


_GCH = 128
_GN = _EW // _GCH
_GTAIL = _EW - _GN * _GCH


def _sc_gather_body(src_hbm, tabs_hbm, dst_hbm, tabd_hbm, gkv_hbm, gq_hbm,
                    idxs, idxd, b1a, b2a, b1b, b2b, gsa, gsb, wsa, wsb):
    wid = lax.axis_index("s") * _NC + lax.axis_index("c")
    base = wid * _EW
    pltpu.sync_copy(src_hbm.at[pl.ds(base, _EW)], idxs)
    pltpu.sync_copy(dst_hbm.at[pl.ds(base, _EW)], idxd)

    def issue(j, b1, b2, gs):
        off = j * _GCH
        pltpu.async_copy(tabs_hbm.at[idxs.at[pl.ds(off, _GCH)]], b1, gs)
        pltpu.async_copy(tabd_hbm.at[idxd.at[pl.ds(off, _GCH)]], b2, gs)

    def wait_g(b1, b2, gs):
        pltpu.make_async_copy(tabs_hbm.at[pl.ds(0, _GCH)], b1, gs).wait()
        pltpu.make_async_copy(tabd_hbm.at[pl.ds(0, _GCH)], b2, gs).wait()

    def wb(j, b1, b2, ws):
        b = base + j * _GCH
        pltpu.async_copy(b1, gkv_hbm.at[pl.ds(b, _GCH)], ws)
        pltpu.async_copy(b2, gq_hbm.at[pl.ds(b, _GCH)], ws)

    def wait_wb(b1, b2, ws):
        pltpu.make_async_copy(b1, gkv_hbm.at[pl.ds(0, _GCH)], ws).wait()
        pltpu.make_async_copy(b2, gq_hbm.at[pl.ds(0, _GCH)], ws).wait()

    issue(0, b1a, b2a, gsa)

    def body(j, carry):
        for pp in range(2):
            if pp == 0:
                b1, b2, gs, ws = b1a, b2a, gsa, wsa
                ob1, ob2, ogs, ows = b1b, b2b, gsb, wsb
            else:
                b1, b2, gs, ws = b1b, b2b, gsb, wsb
                ob1, ob2, ogs, ows = b1a, b2a, gsa, wsa

            @pl.when(lax.rem(j, 2) == pp)
            def _():
                wait_g(b1, b2, gs)

                @pl.when(j >= 1)
                def _():
                    wait_wb(ob1, ob2, ows)

                @pl.when(j + 1 < _GN)
                def _():
                    issue(j + 1, ob1, ob2, ogs)

                wb(j, b1, b2, ws)
        return carry

    lax.fori_loop(0, _GN, body, 0)
    if (_GN - 1) % 2 == 0:
        wait_wb(b1a, b2a, wsa)
    else:
        wait_wb(b1b, b2b, wsb)

    toff = _GN * _GCH
    pltpu.async_copy(tabs_hbm.at[idxs.at[pl.ds(toff, _GTAIL)]],
                     b1a.at[pl.ds(0, _GTAIL)], gsa)
    pltpu.async_copy(tabd_hbm.at[idxd.at[pl.ds(toff, _GTAIL)]],
                     b2a.at[pl.ds(0, _GTAIL)], gsa)
    pltpu.make_async_copy(tabs_hbm.at[pl.ds(0, _GTAIL)],
                          b1a.at[pl.ds(0, _GTAIL)], gsa).wait()
    pltpu.make_async_copy(tabd_hbm.at[pl.ds(0, _GTAIL)],
                          b2a.at[pl.ds(0, _GTAIL)], gsa).wait()
    pltpu.sync_copy(b1a.at[pl.ds(0, _GTAIL)],
                    gkv_hbm.at[pl.ds(base + toff, _GTAIL)])
    pltpu.sync_copy(b2a.at[pl.ds(0, _GTAIL)],
                    gq_hbm.at[pl.ds(base + toff, _GTAIL)])


def _sc_gather(src, tabs, dst, tabd):
    fn = pl.kernel(
        _sc_gather_body,
        out_type=[
            jax.ShapeDtypeStruct((E, 2 * D), jnp.float32),
            jax.ShapeDtypeStruct((E, D), jnp.float32),
        ],
        mesh=plsc.VectorSubcoreMesh(core_axis_name="c", subcore_axis_name="s"),
        scratch_types=[
            pltpu.VMEM((_EW,), jnp.int32),
            pltpu.VMEM((_EW,), jnp.int32),
            pltpu.VMEM((_GCH, 2 * D), jnp.float32),
            pltpu.VMEM((_GCH, D), jnp.float32),
            pltpu.VMEM((_GCH, 2 * D), jnp.float32),
            pltpu.VMEM((_GCH, D), jnp.float32),
            pltpu.SemaphoreType.DMA,
            pltpu.SemaphoreType.DMA,
            pltpu.SemaphoreType.DMA,
            pltpu.SemaphoreType.DMA,
        ],
        compiler_params=pltpu.CompilerParams(needs_layout_passes=False),
    )
    return fn(src, tabs, dst, tabd)


def _sc_scatter_body(pay_hbm, dst_hbm, ea_hbm, zrow_hbm, zs_hbm,
                     outrow_hbm, outs_hbm, ia, ib2, pa, pb2, ea_all, s_tab,
                     shared, lsa, lsb):
    cid = lax.axis_index("c")
    sid = lax.axis_index("s")
    wid = sid * _NC + cid
    base = wid * _EW

    pltpu.sync_copy(zs_hbm, s_tab)
    pltpu.sync_copy(ea_hbm.at[pl.ds(base, _EW)], ea_all)

    @pl.when(sid == 0)
    def _():
        pltpu.sync_copy(zrow_hbm, shared)

    plsc.subcore_barrier()

    def issue(j, ib, pb, ls):
        b = base + j * _CH
        pltpu.async_copy(dst_hbm.at[pl.ds(b, _CH)], ib, ls)
        pltpu.async_copy(pay_hbm.at[pl.ds(b, _CH)], pb, ls)

    def wait_load(ib, pb, ls):
        pltpu.make_async_copy(dst_hbm.at[pl.ds(0, _CH)], ib, ls).wait()
        pltpu.make_async_copy(pay_hbm.at[pl.ds(0, _CH)], pb, ls).wait()

    issue(0, ia, pa, lsa)

    def body(j, carry):
        for pp in range(2):
            if pp == 0:
                ib, pb, ls = ia, pa, lsa
                oib, opb, ols = ib2, pb2, lsb
            else:
                ib, pb, ls = ib2, pb2, lsb
                oib, opb, ols = ia, pa, lsa

            @pl.when(lax.rem(j, 2) == pp)
            def _():
                wait_load(ib, pb, ls)

                @pl.when(j + 1 < _NCH)
                def _():
                    issue(j + 1, oib, opb, ols)

                off = j * _CH
                for j2 in range(_CH // 16):
                    plsc.addupdate_scatter(
                        s_tab, [ib[pl.ds(j2 * 16, 16)]],
                        ea_all[pl.ds(off + j2 * 16, 16)])
                pltpu.sync_copy(pb, shared.at[ib], add=True)
        return carry

    lax.fori_loop(0, _NCH, body, 0)

    pltpu.sync_copy(s_tab, outs_hbm.at[pl.ds(wid * N, N)])
    plsc.subcore_barrier()

    @pl.when(sid < _NS - 1)
    def _():
        pltpu.sync_copy(shared.at[pl.ds(sid * 624, 624)],
                        outrow_hbm.at[cid, pl.ds(sid * 624, 624)])

    @pl.when(sid == _NS - 1)
    def _():
        pltpu.sync_copy(shared.at[pl.ds(15 * 624, N - 15 * 624)],
                        outrow_hbm.at[cid, pl.ds(15 * 624, N - 15 * 624)])


def _sc_scatter(payload, dst, ea, zrow, zs):
    fn = pl.kernel(
        _sc_scatter_body,
        out_type=[
            jax.ShapeDtypeStruct((_NC, N, D), jnp.float32),
            jax.ShapeDtypeStruct((_NW * N,), jnp.float32),
        ],
        mesh=plsc.VectorSubcoreMesh(core_axis_name="c", subcore_axis_name="s"),
        scratch_types=[
            pltpu.VMEM((_CH,), jnp.int32),
            pltpu.VMEM((_CH,), jnp.int32),
            pltpu.VMEM((_CH, D), jnp.float32),
            pltpu.VMEM((_CH, D), jnp.float32),
            pltpu.VMEM((_EW,), jnp.float32),
            pltpu.VMEM((N,), jnp.float32),
            pltpu.VMEM_SHARED((N, D), jnp.float32),
            pltpu.SemaphoreType.DMA,
            pltpu.SemaphoreType.DMA,
        ],
        compiler_params=pltpu.CompilerParams(needs_layout_passes=False),
    )
    return fn(payload, dst, ea, zrow, zs)


_ETILE = 2000
_NEB = E // _ETILE


def _alpha_body(gq_ref, gkv_ref, attr_ref, wep_ref, a_ref, bm_ref):
    qd = gq_ref[...]
    ks = gkv_ref[:, :D]
    qe = jnp.dot(qd, wep_ref[...], preferred_element_type=jnp.float32)
    al = (jnp.sum(qd * ks, axis=1, keepdims=True)
          + jnp.sum(qe * attr_ref[...], axis=1, keepdims=True)
          ) * (1.0 / np.sqrt(float(D)))
    a_ref[...] = al
    bm_ref[...] = jnp.max(al).reshape(1, 1, 1)


def _alpha(gq, gkv, attrp, wep):
    row = lambda i: (i, 0)
    return pl.pallas_call(
        _alpha_body,
        grid=(_NEB,),
        in_specs=[
            pl.BlockSpec((_ETILE, D), row),
            pl.BlockSpec((_ETILE, 2 * D), row),
            pl.BlockSpec((_ETILE, 8), row),
            pl.BlockSpec((D, 8), lambda i: (0, 0)),
        ],
        out_specs=[
            pl.BlockSpec((_ETILE, 1), row),
            pl.BlockSpec((1, 1, 1), lambda i: (i, 0, 0)),
        ],
        out_shape=[
            jax.ShapeDtypeStruct((E, 1), jnp.float32),
            jax.ShapeDtypeStruct((_NEB, 1, 1), jnp.float32),
        ],
        compiler_params=pltpu.CompilerParams(
            dimension_semantics=("parallel",)),
    )(gq, gkv, attrp, wep)


def _payload_body(a_ref, bm_ref, gkv_ref, attr_ref, wet_ref, p_ref, ea_ref):
    kk = jnp.max(bm_ref[...])
    ea = jnp.exp(a_ref[...] - kk)
    vj = gkv_ref[:, D:] + jnp.dot(attr_ref[...], wet_ref[...],
                                  preferred_element_type=jnp.float32)
    p_ref[...] = vj * ea
    ea_ref[...] = ea


def _payload(alpha, bmax, gkv, attrp, wet):
    row = lambda i: (i, 0)
    return pl.pallas_call(
        _payload_body,
        grid=(_NEB,),
        in_specs=[
            pl.BlockSpec((_ETILE, 1), row),
            pl.BlockSpec((_NEB, 1, 1), lambda i: (0, 0, 0)),
            pl.BlockSpec((_ETILE, 2 * D), row),
            pl.BlockSpec((_ETILE, 8), row),
            pl.BlockSpec((8, D), lambda i: (0, 0)),
        ],
        out_specs=[
            pl.BlockSpec((_ETILE, D), row),
            pl.BlockSpec((_ETILE, 1), row),
        ],
        out_shape=[
            jax.ShapeDtypeStruct((E, D), jnp.float32),
            jax.ShapeDtypeStruct((E, 1), jnp.float32),
        ],
        compiler_params=pltpu.CompilerParams(
            dimension_semantics=("parallel",)),
    )(alpha, bmax, gkv, attrp, wet)


def _combine_body(a0_ref, a1_ref, sp_ref, skip_ref, hprev_ref, g_ref,
                  b_ref, out_ref):
    accv = a0_ref[0] + a1_ref[0]
    ones = jnp.ones((_NW, 1), jnp.float32)
    s_col = lax.dot_general(sp_ref[0], ones, (((0,), (0,)), ((), ())),
                            preferred_element_type=jnp.float32)
    msg = accv / (s_col + 1e-16)
    y = hprev_ref[...] + msg + skip_ref[...]
    mu = jnp.mean(y, axis=-1, keepdims=True)
    var = jnp.mean((y - mu) ** 2, axis=-1, keepdims=True)
    out_ref[...] = (y - mu) / jnp.sqrt(var + 1e-5) * g_ref[...] + b_ref[...]


def _combine(acc, spart, skip, hprev, g, b):
    grid = N // _TILE
    row = lambda i: (i, 0)
    fix = lambda i: (0, 0)
    return pl.pallas_call(
        _combine_body,
        grid=(grid,),
        in_specs=[
            pl.BlockSpec((1, _TILE, D), lambda i: (0, i, 0)),
            pl.BlockSpec((1, _TILE, D), lambda i: (1, i, 0)),
            pl.BlockSpec((1, _NW, _TILE), lambda i: (i, 0, 0)),
            pl.BlockSpec((_TILE, D), row),
            pl.BlockSpec((_TILE, D), row),
            pl.BlockSpec((1, D), fix),
            pl.BlockSpec((1, D), fix),
        ],
        out_specs=pl.BlockSpec((_TILE, D), row),
        out_shape=jax.ShapeDtypeStruct((N, D), jnp.float32),
        compiler_params=pltpu.CompilerParams(
            dimension_semantics=("parallel",)),
    )(acc, acc, spart, skip, hprev, g, b)


def _head_body(h_ref, w1_ref, b1_ref, w2_ref, b2_ref, out_ref):
    hid = jax.nn.relu(
        jnp.dot(h_ref[...], w1_ref[...], preferred_element_type=jnp.float32)
        + b1_ref[...])
    w_raw = jnp.dot(hid, w2_ref[...], preferred_element_type=jnp.float32) + b2_ref[0, 0]
    denom = jnp.sum(jnp.abs(w_raw[:, 0:1]))
    out_ref[...] = w_raw / denom


def _head(h, w1, b1, w2, b2):
    return pl.pallas_call(
        _head_body,
        in_specs=[
            pl.BlockSpec((N, D), lambda: (0, 0)),
            pl.BlockSpec((D, D // 2), lambda: (0, 0)),
            pl.BlockSpec((1, D // 2), lambda: (0, 0)),
            pl.BlockSpec((D // 2, 8), lambda: (0, 0)),
            pl.BlockSpec((1, 1), lambda: (0, 0)),
        ],
        out_specs=pl.BlockSpec((N, 8), lambda: (0, 0)),
        out_shape=jax.ShapeDtypeStruct((N, 8), jnp.float32),
    )(h, w1, b1, w2, b2)


def kernel(x, edge_index, edge_attr, params):
    p = params
    src = edge_index[0].astype(jnp.int32)
    dst = edge_index[1].astype(jnp.int32)
    attrp = jnp.pad(edge_attr, ((0, 0), (0, 8 - EDGE_DIM)))
    zrow = jnp.zeros((N, D), jnp.float32)
    zs = jnp.zeros((N,), jnp.float32)

    w_x = p['W_ih'] @ p['W_in']
    b_all = p['b_ih'] + p['b_hh'] + p['W_ih'] @ p['b_in']
    xt2 = jnp.swapaxes(x, 0, 1).reshape(L * N, C)
    gx_all = _gx_matmul(xt2, w_x.T, b_all.reshape(1, -1)).reshape(L, N, 4 * D)
    h = _lstm(gx_all, p['W_hh'].T)

    for l in range(2):
        we = p['We%d' % l]
        wep = jnp.zeros((D, 8), jnp.float32).at[:, :EDGE_DIM].set(we)
        wet = jnp.zeros((8, D), jnp.float32).at[:EDGE_DIM, :].set(we.T)
        tabd, tabs, skip = _projections(
            h, p['Wq%d' % l].T, p['Wk%d' % l].T, p['Wv%d' % l].T,
            p['Wskip%d' % l].T,
            p['bq%d' % l].reshape(1, -1), p['bk%d' % l].reshape(1, -1),
            p['bv%d' % l].reshape(1, -1), p['bskip%d' % l].reshape(1, -1))

        gkv, gq = _sc_gather(src, tabs, dst, tabd)
        alpha, bmax = _alpha(gq, gkv, attrp, wep)
        payload, ea = _payload(alpha, bmax, gkv, attrp, wet)
        acc, spart = _sc_scatter(payload, dst, ea.reshape(E), zrow, zs)
        sp3 = spart.reshape(_NW, N // _TILE, _TILE).swapaxes(0, 1)

        h = _combine(acc, sp3, skip, h,
                     p['ln_g%d' % l].reshape(1, -1),
                     p['ln_b%d' % l].reshape(1, -1))

    out = _head(h, p['W1'].T, p['b1'].reshape(1, -1),
                jnp.zeros((D // 2, 8), jnp.float32).at[:, 0].set(p['W2'][0]),
                p['b2'].reshape(1, 1))
    return out[:, 0]


# speedup vs baseline: 1.3124x; 1.0388x over previous
import functools
import jax
import jax.numpy as jnp
import numpy as np
from jax import lax
from jax.experimental import pallas as pl
from jax.experimental.pallas import tpu as pltpu
from jax.experimental.pallas import tpu_sc as plsc

N = 10000
L = 30
C = 8
D = 128
E = 320000
EDGE_DIM = 2

_TILE = 1000

_NC = 2
_NS = 16
_NW = _NC * _NS
_EW = E // _NW
_CH = 80
_NCH = _EW // _CH


def _gx_body(x_ref, wx_ref, b_ref, out_ref):
    out_ref[...] = (jnp.dot(x_ref[...], wx_ref[...],
                            preferred_element_type=jnp.float32) + b_ref[...])


def _gx_matmul(xt2, wx, b):
    rows = L * N
    tile = 2000
    return pl.pallas_call(
        _gx_body,
        grid=(rows // tile,),
        in_specs=[
            pl.BlockSpec((tile, C), lambda i: (i, 0)),
            pl.BlockSpec((C, 4 * D), lambda i: (0, 0)),
            pl.BlockSpec((1, 4 * D), lambda i: (0, 0)),
        ],
        out_specs=pl.BlockSpec((tile, 4 * D), lambda i: (i, 0)),
        out_shape=jax.ShapeDtypeStruct((rows, 4 * D), jnp.float32),
        compiler_params=pltpu.CompilerParams(
            dimension_semantics=("parallel",)),
    )(xt2, wx, b)


def _lstm_body(gx_ref, whh_ref, out_ref, h_ref, c_ref):
    t = pl.program_id(1)

    @pl.when(t == 0)
    def _():
        h_ref[...] = jnp.zeros_like(h_ref)
        c_ref[...] = jnp.zeros_like(c_ref)

    g = gx_ref[0] + jnp.dot(h_ref[...], whh_ref[...],
                            preferred_element_type=jnp.float32)
    i = jax.nn.sigmoid(g[:, :D])
    f = jax.nn.sigmoid(g[:, D:2 * D])
    gg = jnp.tanh(g[:, 2 * D:3 * D])
    o = jax.nn.sigmoid(g[:, 3 * D:])
    c_ref[...] = f * c_ref[...] + i * gg
    h_ref[...] = o * jnp.tanh(c_ref[...])

    @pl.when(t == L - 1)
    def _():
        out_ref[...] = h_ref[...]


def _lstm(gx_all, whh):
    return pl.pallas_call(
        _lstm_body,
        grid=(N // _TILE, L),
        in_specs=[
            pl.BlockSpec((1, _TILE, 4 * D), lambda n, t: (t, n, 0)),
            pl.BlockSpec((D, 4 * D), lambda n, t: (0, 0)),
        ],
        out_specs=pl.BlockSpec((_TILE, D), lambda n, t: (n, 0)),
        out_shape=jax.ShapeDtypeStruct((N, D), jnp.float32),
        scratch_shapes=[
            pltpu.VMEM((_TILE, D), jnp.float32),
            pltpu.VMEM((_TILE, D), jnp.float32),
        ],
        compiler_params=pltpu.CompilerParams(
            dimension_semantics=("parallel", "arbitrary")),
    )(gx_all, whh)


def _proj_body(h_ref, wq_ref, wk_ref, wv_ref, ws_ref, bq_ref,
               bk_ref, bv_ref, bs_ref, q_ref, tabs_ref, s_ref):
    h = h_ref[...]
    q_ref[...] = jnp.dot(h, wq_ref[...], preferred_element_type=jnp.float32) + bq_ref[...]
    tabs_ref[:, :D] = jnp.dot(h, wk_ref[...], preferred_element_type=jnp.float32) + bk_ref[...]
    tabs_ref[:, D:] = jnp.dot(h, wv_ref[...], preferred_element_type=jnp.float32) + bv_ref[...]
    s_ref[...] = jnp.dot(h, ws_ref[...], preferred_element_type=jnp.float32) + bs_ref[...]


def _projections(h, wq, wk, wv, ws, bq, bk, bv, bs):
    grid = N // _TILE
    row = lambda i: (i, 0)
    fix = lambda i: (0, 0)
    return pl.pallas_call(
        _proj_body,
        grid=(grid,),
        in_specs=[
            pl.BlockSpec((_TILE, D), row),
            pl.BlockSpec((D, D), fix),
            pl.BlockSpec((D, D), fix),
            pl.BlockSpec((D, D), fix),
            pl.BlockSpec((D, D), fix),
            pl.BlockSpec((1, D), fix),
            pl.BlockSpec((1, D), fix),
            pl.BlockSpec((1, D), fix),
            pl.BlockSpec((1, D), fix),
        ],
        out_specs=[
            pl.BlockSpec((_TILE, D), row),
            pl.BlockSpec((_TILE, 2 * D), row),
            pl.BlockSpec((_TILE, D), row),
        ],
        out_shape=[
            jax.ShapeDtypeStruct((N, D), jnp.float32),
            jax.ShapeDtypeStruct((N, 2 * D), jnp.float32),
            jax.ShapeDtypeStruct((N, D), jnp.float32),
        ],
        compiler_params=pltpu.CompilerParams(
            dimension_semantics=("parallel",)),
    )(h, wq, wk, wv, ws, bq, bk, bv, bs)


_GCH = 128
_GN = _EW // _GCH
_GTAIL = _EW - _GN * _GCH


def _sc_gather_body(src_hbm, tabs_hbm, dst_hbm, tabd_hbm, gkv_hbm, gq_hbm,
                    idxs, idxd, b1a, b2a, b1b, b2b, gsa, gsb, wsa, wsb):
    wid = lax.axis_index("s") * _NC + lax.axis_index("c")
    base = wid * _EW
    pltpu.sync_copy(src_hbm.at[pl.ds(base, _EW)], idxs)
    pltpu.sync_copy(dst_hbm.at[pl.ds(base, _EW)], idxd)

    def issue(j, b1, b2, gs):
        off = j * _GCH
        pltpu.async_copy(tabs_hbm.at[idxs.at[pl.ds(off, _GCH)]], b1, gs)
        pltpu.async_copy(tabd_hbm.at[idxd.at[pl.ds(off, _GCH)]], b2, gs)

    def wait_g(b1, b2, gs):
        pltpu.make_async_copy(tabs_hbm.at[pl.ds(0, _GCH)], b1, gs).wait()
        pltpu.make_async_copy(tabd_hbm.at[pl.ds(0, _GCH)], b2, gs).wait()

    def wb(j, b1, b2, ws):
        b = base + j * _GCH
        pltpu.async_copy(b1, gkv_hbm.at[pl.ds(b, _GCH)], ws)
        pltpu.async_copy(b2, gq_hbm.at[pl.ds(b, _GCH)], ws)

    def wait_wb(b1, b2, ws):
        pltpu.make_async_copy(b1, gkv_hbm.at[pl.ds(0, _GCH)], ws).wait()
        pltpu.make_async_copy(b2, gq_hbm.at[pl.ds(0, _GCH)], ws).wait()

    issue(0, b1a, b2a, gsa)

    def body(j, carry):
        for pp in range(2):
            if pp == 0:
                b1, b2, gs, ws = b1a, b2a, gsa, wsa
                ob1, ob2, ogs, ows = b1b, b2b, gsb, wsb
            else:
                b1, b2, gs, ws = b1b, b2b, gsb, wsb
                ob1, ob2, ogs, ows = b1a, b2a, gsa, wsa

            @pl.when(lax.rem(j, 2) == pp)
            def _():
                wait_g(b1, b2, gs)

                @pl.when(j >= 1)
                def _():
                    wait_wb(ob1, ob2, ows)

                @pl.when(j + 1 < _GN)
                def _():
                    issue(j + 1, ob1, ob2, ogs)

                wb(j, b1, b2, ws)
        return carry

    lax.fori_loop(0, _GN, body, 0)
    if (_GN - 1) % 2 == 0:
        wait_wb(b1a, b2a, wsa)
    else:
        wait_wb(b1b, b2b, wsb)

    toff = _GN * _GCH
    pltpu.async_copy(tabs_hbm.at[idxs.at[pl.ds(toff, _GTAIL)]],
                     b1a.at[pl.ds(0, _GTAIL)], gsa)
    pltpu.async_copy(tabd_hbm.at[idxd.at[pl.ds(toff, _GTAIL)]],
                     b2a.at[pl.ds(0, _GTAIL)], gsa)
    pltpu.make_async_copy(tabs_hbm.at[pl.ds(0, _GTAIL)],
                          b1a.at[pl.ds(0, _GTAIL)], gsa).wait()
    pltpu.make_async_copy(tabd_hbm.at[pl.ds(0, _GTAIL)],
                          b2a.at[pl.ds(0, _GTAIL)], gsa).wait()
    pltpu.sync_copy(b1a.at[pl.ds(0, _GTAIL)],
                    gkv_hbm.at[pl.ds(base + toff, _GTAIL)])
    pltpu.sync_copy(b2a.at[pl.ds(0, _GTAIL)],
                    gq_hbm.at[pl.ds(base + toff, _GTAIL)])


def _sc_gather(src, tabs, dst, tabd):
    fn = pl.kernel(
        _sc_gather_body,
        out_type=[
            jax.ShapeDtypeStruct((E, 2 * D), jnp.float32),
            jax.ShapeDtypeStruct((E, D), jnp.float32),
        ],
        mesh=plsc.VectorSubcoreMesh(core_axis_name="c", subcore_axis_name="s"),
        scratch_types=[
            pltpu.VMEM((_EW,), jnp.int32),
            pltpu.VMEM((_EW,), jnp.int32),
            pltpu.VMEM((_GCH, 2 * D), jnp.float32),
            pltpu.VMEM((_GCH, D), jnp.float32),
            pltpu.VMEM((_GCH, 2 * D), jnp.float32),
            pltpu.VMEM((_GCH, D), jnp.float32),
            pltpu.SemaphoreType.DMA,
            pltpu.SemaphoreType.DMA,
            pltpu.SemaphoreType.DMA,
            pltpu.SemaphoreType.DMA,
        ],
        compiler_params=pltpu.CompilerParams(needs_layout_passes=False),
    )
    return fn(src, tabs, dst, tabd)


def _sc_scatter_body(pay_hbm, dst_hbm, ea_hbm, zrow_hbm, zs_hbm,
                     outrow_hbm, outs_hbm, ia, ib2, pa, pb2, ea_all, s_tab,
                     shared, lsa, lsb):
    cid = lax.axis_index("c")
    sid = lax.axis_index("s")
    wid = sid * _NC + cid
    base = wid * _EW

    pltpu.sync_copy(zs_hbm, s_tab)
    pltpu.sync_copy(ea_hbm.at[pl.ds(base, _EW)], ea_all)

    @pl.when(sid == 0)
    def _():
        pltpu.sync_copy(zrow_hbm, shared)

    plsc.subcore_barrier()

    def issue(j, ib, pb, ls):
        b = base + j * _CH
        pltpu.async_copy(dst_hbm.at[pl.ds(b, _CH)], ib, ls)
        pltpu.async_copy(pay_hbm.at[pl.ds(b, _CH)], pb, ls)

    def wait_load(ib, pb, ls):
        pltpu.make_async_copy(dst_hbm.at[pl.ds(0, _CH)], ib, ls).wait()
        pltpu.make_async_copy(pay_hbm.at[pl.ds(0, _CH)], pb, ls).wait()

    issue(0, ia, pa, lsa)

    def body(j, carry):
        for pp in range(2):
            if pp == 0:
                ib, pb, ls = ia, pa, lsa
                oib, opb, ols = ib2, pb2, lsb
            else:
                ib, pb, ls = ib2, pb2, lsb
                oib, opb, ols = ia, pa, lsa

            @pl.when(lax.rem(j, 2) == pp)
            def _():
                wait_load(ib, pb, ls)

                @pl.when(j + 1 < _NCH)
                def _():
                    issue(j + 1, oib, opb, ols)

                off = j * _CH
                for j2 in range(_CH // 16):
                    plsc.addupdate_scatter(
                        s_tab, [ib[pl.ds(j2 * 16, 16)]],
                        ea_all[pl.ds(off + j2 * 16, 16)])
                pltpu.sync_copy(pb, shared.at[ib], add=True)
        return carry

    lax.fori_loop(0, _NCH, body, 0)

    pltpu.sync_copy(s_tab, outs_hbm.at[pl.ds(wid * N, N)])
    plsc.subcore_barrier()

    @pl.when(sid < _NS - 1)
    def _():
        pltpu.sync_copy(shared.at[pl.ds(sid * 624, 624)],
                        outrow_hbm.at[cid, pl.ds(sid * 624, 624)])

    @pl.when(sid == _NS - 1)
    def _():
        pltpu.sync_copy(shared.at[pl.ds(15 * 624, N - 15 * 624)],
                        outrow_hbm.at[cid, pl.ds(15 * 624, N - 15 * 624)])


def _sc_scatter(payload, dst, ea, zrow, zs):
    fn = pl.kernel(
        _sc_scatter_body,
        out_type=[
            jax.ShapeDtypeStruct((_NC, N, D), jnp.float32),
            jax.ShapeDtypeStruct((_NW * N,), jnp.float32),
        ],
        mesh=plsc.VectorSubcoreMesh(core_axis_name="c", subcore_axis_name="s"),
        scratch_types=[
            pltpu.VMEM((_CH,), jnp.int32),
            pltpu.VMEM((_CH,), jnp.int32),
            pltpu.VMEM((_CH, D), jnp.float32),
            pltpu.VMEM((_CH, D), jnp.float32),
            pltpu.VMEM((_EW,), jnp.float32),
            pltpu.VMEM((N,), jnp.float32),
            pltpu.VMEM_SHARED((N, D), jnp.float32),
            pltpu.SemaphoreType.DMA,
            pltpu.SemaphoreType.DMA,
        ],
        compiler_params=pltpu.CompilerParams(needs_layout_passes=False),
    )
    return fn(payload, dst, ea, zrow, zs)


_ETILE = 2000
_NEB = E // _ETILE


def _alpha_body(gq_ref, gkv_ref, attr_ref, wep_ref, a_ref, bm_ref):
    qd = gq_ref[...]
    ks = gkv_ref[...]
    qe = jnp.dot(qd, wep_ref[...], preferred_element_type=jnp.float32)
    al = (jnp.sum(qd * ks, axis=1, keepdims=True)
          + jnp.sum(qe * attr_ref[...], axis=1, keepdims=True)
          ) * (1.0 / np.sqrt(float(D)))
    a_ref[...] = al
    bm_ref[...] = jnp.max(al).reshape(1, 1, 1)


def _alpha(gq, gkv, attrp, wep):
    row = lambda i: (i, 0)
    return pl.pallas_call(
        _alpha_body,
        grid=(_NEB,),
        in_specs=[
            pl.BlockSpec((_ETILE, D), row),
            pl.BlockSpec((_ETILE, D), lambda i: (i, 0)),
            pl.BlockSpec((_ETILE, 8), row),
            pl.BlockSpec((D, 8), lambda i: (0, 0)),
        ],
        out_specs=[
            pl.BlockSpec((_ETILE, 1), row),
            pl.BlockSpec((1, 1, 1), lambda i: (i, 0, 0)),
        ],
        out_shape=[
            jax.ShapeDtypeStruct((E, 1), jnp.float32),
            jax.ShapeDtypeStruct((_NEB, 1, 1), jnp.float32),
        ],
        compiler_params=pltpu.CompilerParams(
            dimension_semantics=("parallel",)),
    )(gq, gkv, attrp, wep)


def _payload_body(a_ref, bm_ref, gkv_ref, attr_ref, wet_ref, p_ref, ea_ref):
    kk = jnp.max(bm_ref[...])
    ea = jnp.exp(a_ref[...] - kk)
    vj = gkv_ref[...] + jnp.dot(attr_ref[...], wet_ref[...],
                                preferred_element_type=jnp.float32)
    p_ref[...] = vj * ea
    ea_ref[...] = ea


def _payload(alpha, bmax, gkv, attrp, wet):
    row = lambda i: (i, 0)
    return pl.pallas_call(
        _payload_body,
        grid=(_NEB,),
        in_specs=[
            pl.BlockSpec((_ETILE, 1), row),
            pl.BlockSpec((_NEB, 1, 1), lambda i: (0, 0, 0)),
            pl.BlockSpec((_ETILE, D), lambda i: (i, 1)),
            pl.BlockSpec((_ETILE, 8), row),
            pl.BlockSpec((8, D), lambda i: (0, 0)),
        ],
        out_specs=[
            pl.BlockSpec((_ETILE, D), row),
            pl.BlockSpec((_ETILE, 1), row),
        ],
        out_shape=[
            jax.ShapeDtypeStruct((E, D), jnp.float32),
            jax.ShapeDtypeStruct((E, 1), jnp.float32),
        ],
        compiler_params=pltpu.CompilerParams(
            dimension_semantics=("parallel",)),
    )(alpha, bmax, gkv, attrp, wet)


def _combine_body(a0_ref, a1_ref, sp_ref, skip_ref, hprev_ref, g_ref,
                  b_ref, out_ref):
    accv = a0_ref[0] + a1_ref[0]
    ones = jnp.ones((_NW, 1), jnp.float32)
    s_col = lax.dot_general(sp_ref[0], ones, (((0,), (0,)), ((), ())),
                            preferred_element_type=jnp.float32)
    msg = accv / (s_col + 1e-16)
    y = hprev_ref[...] + msg + skip_ref[...]
    mu = jnp.mean(y, axis=-1, keepdims=True)
    var = jnp.mean((y - mu) ** 2, axis=-1, keepdims=True)
    out_ref[...] = (y - mu) / jnp.sqrt(var + 1e-5) * g_ref[...] + b_ref[...]


def _combine(acc, spart, skip, hprev, g, b):
    grid = N // _TILE
    row = lambda i: (i, 0)
    fix = lambda i: (0, 0)
    return pl.pallas_call(
        _combine_body,
        grid=(grid,),
        in_specs=[
            pl.BlockSpec((1, _TILE, D), lambda i: (0, i, 0)),
            pl.BlockSpec((1, _TILE, D), lambda i: (1, i, 0)),
            pl.BlockSpec((1, _NW, _TILE), lambda i: (i, 0, 0)),
            pl.BlockSpec((_TILE, D), row),
            pl.BlockSpec((_TILE, D), row),
            pl.BlockSpec((1, D), fix),
            pl.BlockSpec((1, D), fix),
        ],
        out_specs=pl.BlockSpec((_TILE, D), row),
        out_shape=jax.ShapeDtypeStruct((N, D), jnp.float32),
        compiler_params=pltpu.CompilerParams(
            dimension_semantics=("parallel",)),
    )(acc, acc, spart, skip, hprev, g, b)


def _head_body(h_ref, w1_ref, b1_ref, w2_ref, b2_ref, out_ref):
    hid = jax.nn.relu(
        jnp.dot(h_ref[...], w1_ref[...], preferred_element_type=jnp.float32)
        + b1_ref[...])
    w_raw = jnp.dot(hid, w2_ref[...], preferred_element_type=jnp.float32) + b2_ref[0, 0]
    denom = jnp.sum(jnp.abs(w_raw[:, 0:1]))
    out_ref[...] = w_raw / denom


def _head(h, w1, b1, w2, b2):
    return pl.pallas_call(
        _head_body,
        in_specs=[
            pl.BlockSpec((N, D), lambda: (0, 0)),
            pl.BlockSpec((D, D // 2), lambda: (0, 0)),
            pl.BlockSpec((1, D // 2), lambda: (0, 0)),
            pl.BlockSpec((D // 2, 8), lambda: (0, 0)),
            pl.BlockSpec((1, 1), lambda: (0, 0)),
        ],
        out_specs=pl.BlockSpec((N, 8), lambda: (0, 0)),
        out_shape=jax.ShapeDtypeStruct((N, 8), jnp.float32),
    )(h, w1, b1, w2, b2)


def kernel(x, edge_index, edge_attr, params):
    p = params
    src = edge_index[0].astype(jnp.int32)
    dst = edge_index[1].astype(jnp.int32)
    attrp = jnp.pad(edge_attr, ((0, 0), (0, 8 - EDGE_DIM)))
    zrow = jnp.zeros((N, D), jnp.float32)
    zs = jnp.zeros((N,), jnp.float32)

    w_x = p['W_ih'] @ p['W_in']
    b_all = p['b_ih'] + p['b_hh'] + p['W_ih'] @ p['b_in']
    xt2 = jnp.swapaxes(x, 0, 1).reshape(L * N, C)
    gx_all = _gx_matmul(xt2, w_x.T, b_all.reshape(1, -1)).reshape(L, N, 4 * D)
    h = _lstm(gx_all, p['W_hh'].T)

    for l in range(2):
        we = p['We%d' % l]
        wep = jnp.zeros((D, 8), jnp.float32).at[:, :EDGE_DIM].set(we)
        wet = jnp.zeros((8, D), jnp.float32).at[:EDGE_DIM, :].set(we.T)
        tabd, tabs, skip = _projections(
            h, p['Wq%d' % l].T, p['Wk%d' % l].T, p['Wv%d' % l].T,
            p['Wskip%d' % l].T,
            p['bq%d' % l].reshape(1, -1), p['bk%d' % l].reshape(1, -1),
            p['bv%d' % l].reshape(1, -1), p['bskip%d' % l].reshape(1, -1))

        gkv, gq = _sc_gather(src, tabs, dst, tabd)
        alpha, bmax = _alpha(gq, gkv, attrp, wep)
        payload, ea = _payload(alpha, bmax, gkv, attrp, wet)
        acc, spart = _sc_scatter(payload, dst, ea.reshape(E), zrow, zs)
        sp3 = spart.reshape(_NW, N // _TILE, _TILE).swapaxes(0, 1)

        h = _combine(acc, sp3, skip, h,
                     p['ln_g%d' % l].reshape(1, -1),
                     p['ln_b%d' % l].reshape(1, -1))

    out = _head(h, p['W1'].T, p['b1'].reshape(1, -1),
                jnp.zeros((D // 2, 8), jnp.float32).at[:, 0].set(p['W2'][0]),
                p['b2'].reshape(1, 1))
    return out[:, 0]
